# baseline (speedup 1.0000x reference)
import functools
import jax
import jax.numpy as jnp
from jax import lax
from jax.experimental import pallas as pl
from jax.experimental.pallas import tpu as pltpu
from jax.experimental.pallas import tpu_sc as plsc

_NC = 2
_NS = 16
_NW = _NC * _NS
_NIG = _NW // 2
_L = 16


def _body(chunk, nchunks, b_per_ig, table_hbm, idx_hbm, out_hbm,
          ttile, idx0, idx1, stag0, stag1,
          sem_t, sem_i0, sem_i1, sem_w0, sem_w1):
    V, D = table_hbm.shape
    H = D // 2
    wid = lax.axis_index("s") * _NC + lax.axis_index("c")
    ig = wid // 2
    h = wid % 2
    base = ig * b_per_ig
    idxb = (idx0, idx1)
    stag = (stag0, stag1)
    sem_i = (sem_i0, sem_i1)
    sem_w = (sem_w0, sem_w1)

    pltpu.async_copy(table_hbm.at[:, pl.ds(h * H, H)], ttile, sem_t).wait()

    def start_idx(c, b):
        pltpu.async_copy(idx_hbm.at[pl.ds(base + c * chunk, chunk)],
                         idxb[b], sem_i[b])

    def wait_idx(c, b):
        pltpu.make_async_copy(idx_hbm.at[pl.ds(base + c * chunk, chunk)],
                              idxb[b], sem_i[b]).wait()

    def start_write(c, b):
        pltpu.async_copy(
            stag[b], out_hbm.at[pl.ds(base + c * chunk, chunk),
                                pl.ds(h * H, H)], sem_w[b])

    def wait_write(c, b):
        pltpu.make_async_copy(
            stag[b], out_hbm.at[pl.ds(base + c * chunk, chunk),
                                pl.ds(h * H, H)], sem_w[b]).wait()

    lane = lax.iota(jnp.int32, _L)
    cols = [jnp.full((_L,), d, jnp.int32) for d in range(H)]

    def compute(b):
        def group(r, carry):
            iv = idxb[b][pl.ds(r * _L, _L)]
            rv = lane + r * _L
            for d in range(H):
                vals = plsc.load_gather(ttile, [iv, cols[d]])
                plsc.store_scatter(stag[b], [rv, cols[d]], vals)
            return carry
        lax.fori_loop(0, chunk // _L, group, 0)

    for b in range(2):
        start_idx(b, b)

    def pair(g, carry):
        for b in range(2):
            c = g * 2 + b
            wait_idx(c, b)
            compute(b)
            start_write(c, b)
            wait_write(c, b)
            start_idx(c + 2, b)
        return carry

    npairs = nchunks // 2
    lax.fori_loop(0, npairs - 1, pair, 0)

    for b in range(2):
        c = (npairs - 1) * 2 + b
        wait_idx(c, b)
        compute(b)
        start_write(c, b)
    for b in range(2):
        c = (npairs - 1) * 2 + b
        wait_write(c, b)


def kernel(input_pos_tensors, table):
    B0, T = input_pos_tensors.shape
    V, D = table.shape
    B = B0 * T
    idx = input_pos_tensors.reshape(B).astype(jnp.int32)

    b_per_ig = B // _NIG
    chunk = 512
    nchunks = b_per_ig // chunk
    H = D // 2

    mesh = plsc.VectorSubcoreMesh(
        core_axis_name="c", subcore_axis_name="s",
        num_cores=_NC, num_subcores=_NS)
    run = pl.kernel(
        functools.partial(_body, chunk, nchunks, b_per_ig),
        out_type=jax.ShapeDtypeStruct((B, D), jnp.float32),
        mesh=mesh,
        scratch_types=[
            pltpu.VMEM((V, H), jnp.float32),
            pltpu.VMEM((chunk,), jnp.int32),
            pltpu.VMEM((chunk,), jnp.int32),
            pltpu.VMEM((chunk, H), jnp.float32),
            pltpu.VMEM((chunk, H), jnp.float32),
            pltpu.SemaphoreType.DMA,
            pltpu.SemaphoreType.DMA,
            pltpu.SemaphoreType.DMA,
            pltpu.SemaphoreType.DMA,
            pltpu.SemaphoreType.DMA,
        ],
        compiler_params=pltpu.CompilerParams(
            use_tc_tiling_on_sc=False, needs_layout_passes=False),
    )
    out = run(table, idx)
    return out.reshape(B0, T, D)


# speedup vs baseline: 1.2877x; 1.2877x over previous
import functools
import jax
import jax.numpy as jnp
from jax import lax
from jax.experimental import pallas as pl
from jax.experimental.pallas import tpu as pltpu
from jax.experimental.pallas import tpu_sc as plsc

_NC = 2
_NS = 16
_NW = _NC * _NS
_NIG = _NW // 2
_L = 16


def _body(chunk, nchunks, b_per_ig, table_hbm, idx_hbm, out_hbm,
          ttile, idx0, idx1, stag0, stag1,
          sem_t, sem_i0, sem_i1, sem_w0, sem_w1):
    V, D = table_hbm.shape
    H = D // 2
    wid = lax.axis_index("s") * _NC + lax.axis_index("c")
    ig = wid // 2
    h = wid % 2
    base = ig * b_per_ig
    idxb = (idx0, idx1)
    stag = (stag0, stag1)
    sem_i = (sem_i0, sem_i1)
    sem_w = (sem_w0, sem_w1)

    pltpu.async_copy(table_hbm.at[:, pl.ds(h * H, H)], ttile, sem_t).wait()

    def start_idx(c, b):
        pltpu.async_copy(idx_hbm.at[pl.ds(base + c * chunk, chunk)],
                         idxb[b], sem_i[b])

    def wait_idx(c, b):
        pltpu.make_async_copy(idx_hbm.at[pl.ds(base + c * chunk, chunk)],
                              idxb[b], sem_i[b]).wait()

    def start_write(c, b):
        pltpu.async_copy(
            stag[b], out_hbm.at[pl.ds(base + c * chunk, chunk),
                                pl.ds(h * H, H)], sem_w[b])

    def wait_write(c, b):
        pltpu.make_async_copy(
            stag[b], out_hbm.at[pl.ds(base + c * chunk, chunk),
                                pl.ds(h * H, H)], sem_w[b]).wait()

    lane = lax.iota(jnp.int32, _L)
    cols = [jnp.full((_L,), d, jnp.int32) for d in range(H)]

    def compute(b):
        @plsc.parallel_loop(0, chunk // _L, unroll=2)
        def group(r):
            iv = idxb[b][pl.ds(r * _L, _L)]
            rv = lane + r * _L
            vals = [plsc.load_gather(ttile, [iv, cols[d]]) for d in range(H)]
            for d in range(H):
                plsc.store_scatter(stag[b], [rv, cols[d]], vals[d])

    for b in range(2):
        start_idx(b, b)

    def pair(g, carry):
        for b in range(2):
            c = g * 2 + b
            wait_idx(c, b)
            compute(b)
            start_write(c, b)
            wait_write(c, b)
            start_idx(c + 2, b)
        return carry

    npairs = nchunks // 2
    lax.fori_loop(0, npairs - 1, pair, 0)

    for b in range(2):
        c = (npairs - 1) * 2 + b
        wait_idx(c, b)
        compute(b)
        start_write(c, b)
    for b in range(2):
        c = (npairs - 1) * 2 + b
        wait_write(c, b)


def kernel(input_pos_tensors, table):
    B0, T = input_pos_tensors.shape
    V, D = table.shape
    B = B0 * T
    idx = input_pos_tensors.reshape(B).astype(jnp.int32)

    b_per_ig = B // _NIG
    chunk = 512
    nchunks = b_per_ig // chunk
    H = D // 2

    mesh = plsc.VectorSubcoreMesh(
        core_axis_name="c", subcore_axis_name="s",
        num_cores=_NC, num_subcores=_NS)
    run = pl.kernel(
        functools.partial(_body, chunk, nchunks, b_per_ig),
        out_type=jax.ShapeDtypeStruct((B, D), jnp.float32),
        mesh=mesh,
        scratch_types=[
            pltpu.VMEM((V, H), jnp.float32),
            pltpu.VMEM((chunk,), jnp.int32),
            pltpu.VMEM((chunk,), jnp.int32),
            pltpu.VMEM((chunk, H), jnp.float32),
            pltpu.VMEM((chunk, H), jnp.float32),
            pltpu.SemaphoreType.DMA,
            pltpu.SemaphoreType.DMA,
            pltpu.SemaphoreType.DMA,
            pltpu.SemaphoreType.DMA,
            pltpu.SemaphoreType.DMA,
        ],
        compiler_params=pltpu.CompilerParams(
            use_tc_tiling_on_sc=False, needs_layout_passes=False),
    )
    out = run(table, idx)
    return out.reshape(B0, T, D)


# speedup vs baseline: 1.2912x; 1.0027x over previous
import functools
import jax
import jax.numpy as jnp
from jax import lax
from jax.experimental import pallas as pl
from jax.experimental.pallas import tpu as pltpu
from jax.experimental.pallas import tpu_sc as plsc

_NC = 2
_NS = 16
_NW = _NC * _NS
_NIG = _NW // 2
_L = 16


def _body(chunk, nchunks, b_per_ig, table_hbm, idx_hbm, out_hbm,
          ttile, idx0, idx1, stag0, stag1,
          sem_t, sem_i0, sem_i1, sem_w0, sem_w1):
    V, D = table_hbm.shape
    H = D // 2
    wid = lax.axis_index("s") * _NC + lax.axis_index("c")
    ig = wid // 2
    h = wid % 2
    base = ig * b_per_ig
    idxb = (idx0, idx1)
    stag = (stag0, stag1)
    sem_i = (sem_i0, sem_i1)
    sem_w = (sem_w0, sem_w1)

    pltpu.async_copy(table_hbm.at[:, pl.ds(h * H, H)], ttile, sem_t).wait()

    def start_idx(c, b):
        pltpu.async_copy(idx_hbm.at[pl.ds(base + c * chunk, chunk)],
                         idxb[b], sem_i[b])

    def wait_idx(c, b):
        pltpu.make_async_copy(idx_hbm.at[pl.ds(base + c * chunk, chunk)],
                              idxb[b], sem_i[b]).wait()

    def start_write(c, b):
        pltpu.async_copy(
            stag[b], out_hbm.at[pl.ds(base + c * chunk, chunk),
                                pl.ds(h * H, H)], sem_w[b])

    def wait_write(c, b):
        pltpu.make_async_copy(
            stag[b], out_hbm.at[pl.ds(base + c * chunk, chunk),
                                pl.ds(h * H, H)], sem_w[b]).wait()

    lane = lax.iota(jnp.int32, _L)
    cols = [jnp.full((_L,), d, jnp.int32) for d in range(H)]

    def compute(b):
        @plsc.parallel_loop(0, chunk // _L, unroll=2)
        def group(r):
            iv = idxb[b][pl.ds(r * _L, _L)]
            rv = lane + r * _L
            vals = [plsc.load_gather(ttile, [iv, cols[d]]) for d in range(H)]
            for d in range(H):
                plsc.store_scatter(stag[b], [rv, cols[d]], vals[d])

    for b in range(2):
        start_idx(b, b)

    def pair(g, carry):
        for b in range(2):
            c = g * 2 + b
            wait_idx(c, b)
            compute(b)
            start_write(c, b)
            wait_write(c, b)
            start_idx(c + 2, b)
        return carry

    npairs = nchunks // 2
    lax.fori_loop(0, npairs - 1, pair, 0)

    for b in range(2):
        c = (npairs - 1) * 2 + b
        wait_idx(c, b)
        compute(b)
        start_write(c, b)
    for b in range(2):
        c = (npairs - 1) * 2 + b
        wait_write(c, b)


def kernel(input_pos_tensors, table):
    B0, T = input_pos_tensors.shape
    V, D = table.shape
    B = B0 * T
    idx = input_pos_tensors.reshape(B).astype(jnp.int32)

    b_per_ig = B // _NIG
    chunk = 512
    nchunks = b_per_ig // chunk
    H = D // 2

    mesh = plsc.VectorSubcoreMesh(
        core_axis_name="c", subcore_axis_name="s",
        num_cores=_NC, num_subcores=_NS)
    run = pl.kernel(
        functools.partial(_body, chunk, nchunks, b_per_ig),
        out_type=jax.ShapeDtypeStruct((B, D), jnp.float32),
        mesh=mesh,
        scratch_types=[
            pltpu.VMEM((V, H), jnp.float32),
            pltpu.VMEM((chunk,), jnp.int32),
            pltpu.VMEM((chunk,), jnp.int32),
            pltpu.VMEM((chunk, H), jnp.float32),
            pltpu.VMEM((chunk, H), jnp.float32),
            pltpu.SemaphoreType.DMA,
            pltpu.SemaphoreType.DMA,
            pltpu.SemaphoreType.DMA,
            pltpu.SemaphoreType.DMA,
            pltpu.SemaphoreType.DMA,
        ],
        compiler_params=pltpu.CompilerParams(
            use_tc_tiling_on_sc=False, needs_layout_passes=False,
            disable_bounds_checks=True),
    )
    out = run(table, idx)
    return out.reshape(B0, T, D)


# speedup vs baseline: 3.6923x; 2.8597x over previous
import functools
import jax
import jax.numpy as jnp
from jax import lax
from jax.experimental import pallas as pl
from jax.experimental.pallas import tpu as pltpu
from jax.experimental.pallas import tpu_sc as plsc

_NC = 2
_NS = 16
_NW = _NC * _NS
_NIG = _NW // 2
_L = 16


def _body(chunk, nchunks, b_per_ig, table_hbm, idx_hbm, out_hbm,
          ttile, idx0, idx1, stag0, stag1,
          sem_t, sem_i0, sem_i1, sem_w0, sem_w1):
    V, D = table_hbm.shape
    H = D // 2
    wid = lax.axis_index("s") * _NC + lax.axis_index("c")
    ig = wid // 2
    h = wid % 2
    base = ig * b_per_ig
    idxb = (idx0, idx1)
    stag = (stag0, stag1)
    sem_i = (sem_i0, sem_i1)
    sem_w = (sem_w0, sem_w1)

    pltpu.async_copy(table_hbm.at[:, pl.ds(h * H, H)], ttile, sem_t).wait()

    def start_idx(c, b):
        pltpu.async_copy(idx_hbm.at[pl.ds(base + c * chunk, chunk)],
                         idxb[b], sem_i[b])

    def wait_idx(c, b):
        pltpu.make_async_copy(idx_hbm.at[pl.ds(base + c * chunk, chunk)],
                              idxb[b], sem_i[b]).wait()

    def start_write(c, b):
        pltpu.async_copy(
            stag[b], out_hbm.at[pl.ds(base + c * chunk, chunk),
                                pl.ds(h * H, H)], sem_w[b])

    def wait_write(c, b):
        pltpu.make_async_copy(
            stag[b], out_hbm.at[pl.ds(base + c * chunk, chunk),
                                pl.ds(h * H, H)], sem_w[b]).wait()

    def compute(b):
        @plsc.parallel_loop(0, chunk // _L, unroll=2)
        def group(r):
            iv = idxb[b][pl.ds(r * _L, _L)]
            for l in range(_L):
                s = iv[l]
                for d in range(0, H, _L):
                    stag[b][r * _L + l, pl.ds(d, _L)] = ttile[s, pl.ds(d, _L)]

    for b in range(2):
        start_idx(b, b)

    def pair(g, carry):
        for b in range(2):
            c = g * 2 + b
            wait_idx(c, b)
            compute(b)
            start_write(c, b)
            wait_write(c, b)
            start_idx(c + 2, b)
        return carry

    npairs = nchunks // 2
    lax.fori_loop(0, npairs - 1, pair, 0)

    for b in range(2):
        c = (npairs - 1) * 2 + b
        wait_idx(c, b)
        compute(b)
        start_write(c, b)
    for b in range(2):
        c = (npairs - 1) * 2 + b
        wait_write(c, b)


def kernel(input_pos_tensors, table):
    B0, T = input_pos_tensors.shape
    V, D = table.shape
    B = B0 * T
    idx = input_pos_tensors.reshape(B).astype(jnp.int32)

    b_per_ig = B // _NIG
    chunk = 512
    nchunks = b_per_ig // chunk
    H = D // 2

    mesh = plsc.VectorSubcoreMesh(
        core_axis_name="c", subcore_axis_name="s",
        num_cores=_NC, num_subcores=_NS)
    run = pl.kernel(
        functools.partial(_body, chunk, nchunks, b_per_ig),
        out_type=jax.ShapeDtypeStruct((B, D), jnp.float32),
        mesh=mesh,
        scratch_types=[
            pltpu.VMEM((V, H), jnp.float32),
            pltpu.VMEM((chunk,), jnp.int32),
            pltpu.VMEM((chunk,), jnp.int32),
            pltpu.VMEM((chunk, H), jnp.float32),
            pltpu.VMEM((chunk, H), jnp.float32),
            pltpu.SemaphoreType.DMA,
            pltpu.SemaphoreType.DMA,
            pltpu.SemaphoreType.DMA,
            pltpu.SemaphoreType.DMA,
            pltpu.SemaphoreType.DMA,
        ],
        compiler_params=pltpu.CompilerParams(
            use_tc_tiling_on_sc=False, needs_layout_passes=False,
            disable_bounds_checks=True),
    )
    out = run(table, idx)
    return out.reshape(B0, T, D)


# speedup vs baseline: 3.7778x; 1.0232x over previous
import functools
import jax
import jax.numpy as jnp
from jax import lax
from jax.experimental import pallas as pl
from jax.experimental.pallas import tpu as pltpu
from jax.experimental.pallas import tpu_sc as plsc

_NC = 2
_NS = 16
_NW = _NC * _NS
_NIG = _NW // 2
_L = 16


def _body(chunk, nchunks, b_per_ig, table_hbm, idx_hbm, out_hbm,
          ttile, idx0, idx1, stag0, stag1,
          sem_t, sem_i0, sem_i1, sem_w0, sem_w1):
    V, D = table_hbm.shape
    H = D // 2
    wid = lax.axis_index("s") * _NC + lax.axis_index("c")
    ig = wid // 2
    h = wid % 2
    base = ig * b_per_ig
    idxb = (idx0, idx1)
    stag = (stag0, stag1)
    sem_i = (sem_i0, sem_i1)
    sem_w = (sem_w0, sem_w1)

    pltpu.async_copy(table_hbm.at[:, pl.ds(h * H, H)], ttile, sem_t).wait()

    def start_idx(c, b):
        pltpu.async_copy(idx_hbm.at[pl.ds(base + c * chunk, chunk)],
                         idxb[b], sem_i[b])

    def wait_idx(c, b):
        pltpu.make_async_copy(idx_hbm.at[pl.ds(base + c * chunk, chunk)],
                              idxb[b], sem_i[b]).wait()

    def start_write(c, b):
        pltpu.async_copy(
            stag[b], out_hbm.at[pl.ds(base + c * chunk, chunk),
                                pl.ds(h * H, H)], sem_w[b])

    def wait_write(c, b):
        pltpu.make_async_copy(
            stag[b], out_hbm.at[pl.ds(base + c * chunk, chunk),
                                pl.ds(h * H, H)], sem_w[b]).wait()

    def compute(b):
        @plsc.parallel_loop(0, chunk // _L, unroll=2)
        def group(r):
            iv = idxb[b][pl.ds(r * _L, _L)]
            ss = [iv[l] for l in range(_L)]
            vals = [[ttile[ss[l], pl.ds(d, _L)] for d in range(0, H, _L)]
                    for l in range(_L)]
            for l in range(_L):
                for k, d in enumerate(range(0, H, _L)):
                    stag[b][r * _L + l, pl.ds(d, _L)] = vals[l][k]

    for b in range(2):
        start_idx(b, b)

    def pair(g, carry):
        for b in range(2):
            c = g * 2 + b
            wait_idx(c, b)
            compute(b)
            start_write(c, b)
            wait_write(c, b)
            start_idx(c + 2, b)
        return carry

    npairs = nchunks // 2
    lax.fori_loop(0, npairs - 1, pair, 0)

    for b in range(2):
        c = (npairs - 1) * 2 + b
        wait_idx(c, b)
        compute(b)
        start_write(c, b)
    for b in range(2):
        c = (npairs - 1) * 2 + b
        wait_write(c, b)


def kernel(input_pos_tensors, table):
    B0, T = input_pos_tensors.shape
    V, D = table.shape
    B = B0 * T
    idx = input_pos_tensors.reshape(B).astype(jnp.int32)

    b_per_ig = B // _NIG
    chunk = 512
    nchunks = b_per_ig // chunk
    H = D // 2

    mesh = plsc.VectorSubcoreMesh(
        core_axis_name="c", subcore_axis_name="s",
        num_cores=_NC, num_subcores=_NS)
    run = pl.kernel(
        functools.partial(_body, chunk, nchunks, b_per_ig),
        out_type=jax.ShapeDtypeStruct((B, D), jnp.float32),
        mesh=mesh,
        scratch_types=[
            pltpu.VMEM((V, H), jnp.float32),
            pltpu.VMEM((chunk,), jnp.int32),
            pltpu.VMEM((chunk,), jnp.int32),
            pltpu.VMEM((chunk, H), jnp.float32),
            pltpu.VMEM((chunk, H), jnp.float32),
            pltpu.SemaphoreType.DMA,
            pltpu.SemaphoreType.DMA,
            pltpu.SemaphoreType.DMA,
            pltpu.SemaphoreType.DMA,
            pltpu.SemaphoreType.DMA,
        ],
        compiler_params=pltpu.CompilerParams(
            use_tc_tiling_on_sc=False, needs_layout_passes=False,
            disable_bounds_checks=True),
    )
    out = run(table, idx)
    return out.reshape(B0, T, D)
